# Initial kernel scaffold; baseline (speedup 1.0000x reference)
import jax
import jax.numpy as jnp
from jax.experimental import pallas as pl


def kernel(tokens, emb, wqkv, bqkv, wo, bo, ln1g, ln1b, ln2g, ln2b, w1, b1, w2, b2, wg, ew1, eb1, ew2, eb2, lnfg, lnfb):
    raise NotImplementedError("write your pallas kernel here")


# speedup vs baseline: 1.0999x; 1.0999x over previous
import functools
import math
import jax
import jax.numpy as jnp
from jax import lax
from jax.experimental import pallas as pl
from jax.experimental.pallas import tpu as pltpu
from jax.experimental.pallas import tpu_sc as plsc

_NEG = float(-1e9)


def _ln(x, g, b):
    mu = jnp.mean(x, axis=-1, keepdims=True)
    var = jnp.mean((x - mu) * (x - mu), axis=-1, keepdims=True)
    return (x - mu) * jax.lax.rsqrt(var + 1e-5) * g + b


def _scale_body(x_ref, o_ref, *, s):
    o_ref[...] = x_ref[...] * jnp.float32(s)


def _ln_qkv_body(x_ref, g_ref, b_ref, w_ref, bias_ref, o_ref):
    h = _ln(x_ref[...], g_ref[...], b_ref[...])
    o_ref[...] = (
        jnp.dot(h, w_ref[...], preferred_element_type=jnp.float32)
        + bias_ref[...]
    )


def _attn_body(q_ref, k_ref, v_ref, o_ref, *, qb_size, dh):
    qb = pl.program_id(1)
    for hh in range(2):
        sl = slice(hh * dh, (hh + 1) * dh)
        q = q_ref[:, sl]
        k = k_ref[:, sl]
        s = jax.lax.dot_general(
            q, k, (((1,), (1,)), ((), ())),
            preferred_element_type=jnp.float32,
        ) * (1.0 / math.sqrt(dh))
        rows = qb * qb_size + jax.lax.broadcasted_iota(jnp.int32, s.shape, 0)
        cols = jax.lax.broadcasted_iota(jnp.int32, s.shape, 1)
        s = jnp.where(cols <= rows, s, _NEG)
        m = jnp.max(s, axis=-1, keepdims=True)
        p = jnp.exp(s - m)
        p = p / jnp.sum(p, axis=-1, keepdims=True)
        o_ref[:, sl] = jnp.dot(p, v_ref[:, sl],
                               preferred_element_type=jnp.float32)


def _proj_res_body(o_ref, w_ref, b_ref, r_ref, out_ref, *, res_scale):
    out_ref[...] = (
        jnp.dot(o_ref[...], w_ref[...], preferred_element_type=jnp.float32)
        + b_ref[...]
        + jnp.float32(res_scale) * r_ref[...]
    )


def _ffn_body(x_ref, g_ref, b_ref, w1_ref, b1_ref, w2_ref, b2_ref, o_ref):
    h = _ln(x_ref[...], g_ref[...], b_ref[...])
    h1 = jax.nn.gelu(
        jnp.dot(h, w1_ref[...], preferred_element_type=jnp.float32)
        + b1_ref[...]
    )
    o_ref[...] = (
        jnp.dot(h1, w2_ref[...], preferred_element_type=jnp.float32)
        + b2_ref[...]
        + x_ref[...]
    )


def _route_body(x_ref, g_ref, b_ref, wg_ref,
                h_ref, s1s_ref, s2s_ref, s1g_ref, s2g_ref,
                g1_ref, g2_ref, laux_ref,
                loc1_ref, loc2_ref,
                *, T, E, C, EP):
    x = x_ref[...]
    h = _ln(x, g_ref[...], b_ref[...])
    h_ref[pl.ds(0, T), :] = h
    h_ref[pl.ds(T, h_ref.shape[0] - T), :] = jnp.zeros(
        (h_ref.shape[0] - T, h.shape[1]), jnp.float32)

    logits = jnp.dot(h, wg_ref[...], preferred_element_type=jnp.float32)
    lane = jax.lax.broadcasted_iota(jnp.int32, logits.shape, 1)
    lmask = lane < E
    mx = jnp.max(jnp.where(lmask, logits, _NEG), axis=-1, keepdims=True)
    ex = jnp.where(lmask, jnp.exp(logits - mx), 0.0)
    gates = ex / jnp.sum(ex, axis=-1, keepdims=True)

    mx1 = jnp.max(gates, axis=-1, keepdims=True)
    idx1 = jnp.min(jnp.where((gates == mx1) & lmask, lane, EP),
                   axis=-1, keepdims=True)
    mask1 = (lane == idx1).astype(jnp.float32)
    gates2 = gates * (1.0 - mask1)
    mx2 = jnp.max(gates2, axis=-1, keepdims=True)
    idx2 = jnp.min(jnp.where((gates2 == mx2) & lmask, lane, EP),
                   axis=-1, keepdims=True)
    mask2 = (lane == idx2).astype(jnp.float32)

    count1 = jnp.sum(mask1, axis=0, keepdims=True)

    CH = 128
    r = jax.lax.broadcasted_iota(jnp.int32, (CH, CH), 0)
    c = jax.lax.broadcasted_iota(jnp.int32, (CH, CH), 1)
    tri = (r > c).astype(jnp.float32)

    off1 = jnp.zeros((1, EP), jnp.float32)
    off2 = count1
    for i in range(T // CH):
        b1 = mask1[i * CH:(i + 1) * CH, :]
        b2 = mask2[i * CH:(i + 1) * CH, :]
        loc1_ref[i * CH:(i + 1) * CH, :] = (
            jnp.dot(tri, b1, preferred_element_type=jnp.float32) + off1)
        loc2_ref[i * CH:(i + 1) * CH, :] = (
            jnp.dot(tri, b2, preferred_element_type=jnp.float32) + off2)
        off1 = off1 + jnp.sum(b1, axis=0, keepdims=True)
        off2 = off2 + jnp.sum(b2, axis=0, keepdims=True)

    loc1 = loc1_ref[...]
    loc2 = loc2_ref[...]

    me = jnp.mean(gates, axis=0, keepdims=True)
    ce = jnp.mean(mask1, axis=0, keepdims=True)
    laux_ref[...] = (jnp.sum(me * ce) * jnp.float32(E * E))[None, None]

    mask1c = mask1 * (loc1 < C).astype(jnp.float32)
    mask2c = mask2 * (loc2 < C).astype(jnp.float32)

    g1r = jnp.sum(gates * mask1c, axis=-1, keepdims=True)
    g2r = jnp.sum(gates * mask2c, axis=-1, keepdims=True)
    denom = g1r + g2r
    denom = jnp.where(denom > 0, denom, 1.0)
    g1_ref[...] = g1r / denom
    g2_ref[...] = g2r / denom

    p1 = jnp.sum(loc1 * mask1c, axis=-1, keepdims=True).astype(jnp.int32)
    p2 = jnp.sum(loc2 * mask2c, axis=-1, keepdims=True).astype(jnp.int32)
    kept1 = jnp.sum(mask1c, axis=-1, keepdims=True) > 0
    kept2 = jnp.sum(mask2c, axis=-1, keepdims=True) > 0
    slot1 = idx1 * C + p1
    slot2 = idx2 * C + p2
    trash = jnp.int32(E * C)
    s1s_ref[...] = jnp.where(kept1 & (g1r > 0), slot1, trash)
    s2s_ref[...] = jnp.where(kept2 & (g2r > 0), slot2, trash)
    s1g_ref[...] = jnp.where(kept1, slot1, jnp.int32(E * C))
    s2g_ref[...] = jnp.where(kept2, slot2, jnp.int32(E * C))


def _expert_ffn_body(x_ref, w1_ref, b1_ref, w2_ref, b2_ref, o_ref):
    f = pl.program_id(1)
    h1 = jax.nn.gelu(
        jnp.dot(x_ref[0], w1_ref[0], preferred_element_type=jnp.float32)
        + b1_ref[0]
    )
    part = jnp.dot(h1, w2_ref[0], preferred_element_type=jnp.float32)

    @pl.when(f == 0)
    def _():
        o_ref[0] = part + b2_ref[0]

    @pl.when(f != 0)
    def _():
        o_ref[0] = o_ref[0] + part


def _combine_ln_body(x_ref, e1_ref, e2_ref, g1_ref, g2_ref, g_ref, b_ref,
                     o_ref):
    x = (x_ref[...]
         + g1_ref[...] * e1_ref[...]
         + g2_ref[...] * e2_ref[...])
    o_ref[...] = _ln(x, g_ref[...], b_ref[...])


def _sc_mesh():
    return plsc.VectorSubcoreMesh(core_axis_name="c", subcore_axis_name="s")


def _sc_gather(table, idx, rows_total, d):
    info = plsc.get_sparse_core_info()
    nw = info.num_cores * info.num_subcores
    bpw = rows_total // nw

    @functools.partial(
        pl.kernel,
        mesh=_sc_mesh(),
        out_type=jax.ShapeDtypeStruct((rows_total, d), jnp.float32),
        scratch_types=[
            pltpu.VMEM((bpw,), jnp.int32),
            pltpu.VMEM((bpw, d), jnp.float32),
            pltpu.SemaphoreType.DMA,
        ],
    )
    def k(table_hbm, idx_hbm, out_hbm, idx_v, rows_v, sem):
        wid = lax.axis_index("s") * info.num_cores + lax.axis_index("c")
        base = wid * bpw
        pltpu.sync_copy(idx_hbm.at[pl.ds(base, bpw)], idx_v)
        pltpu.async_copy(table_hbm.at[idx_v], rows_v, sem).wait()
        pltpu.sync_copy(rows_v, out_hbm.at[pl.ds(base, bpw)])

    return k(table, idx)


def _sc_invert(slot1, slot2, arange_t, T, nslots):
    info = plsc.get_sparse_core_info()
    nc = T // 128
    npad = nslots + 16

    @functools.partial(
        pl.kernel,
        mesh=_sc_mesh(),
        out_type=jax.ShapeDtypeStruct((npad,), jnp.int32),
        scratch_types=[
            pltpu.VMEM((npad,), jnp.int32),
            pltpu.VMEM((nc, 128), jnp.int32),
            pltpu.VMEM((nc, 128), jnp.int32),
            pltpu.VMEM((nc, 128), jnp.int32),
            pltpu.SemaphoreType.DMA,
        ],
    )
    def k(s1_hbm, s2_hbm, ar_hbm, out_hbm, st_v, s1_v, s2_v, ar_v, sem):
        wid = lax.axis_index("s") * info.num_cores + lax.axis_index("c")

        @pl.when(wid == 0)
        def _():
            pltpu.sync_copy(s1_hbm, s1_v)
            pltpu.sync_copy(s2_hbm, s2_v)
            pltpu.sync_copy(ar_hbm, ar_v)
            sent = jnp.full((16,), T, jnp.int32)
            for cc in range(npad // 16):
                st_v[pl.ds(cc * 16, 16)] = sent
            pltpu.sync_copy(st_v, out_hbm)
            for cc in range(nc):
                pltpu.async_copy(
                    ar_v.at[cc], out_hbm.at[s1_v.at[cc]], sem).wait()
                pltpu.async_copy(
                    ar_v.at[cc], out_hbm.at[s2_v.at[cc]], sem).wait()

    return k(slot1, slot2, arange_t)[:nslots]


def _attention_block(x, wqkv, bqkv, wo, bo, ln_g, ln_b, res_scale, T, D, H):
    dh = D // H
    SB = 256
    QB = 512
    qkv = pl.pallas_call(
        _ln_qkv_body,
        grid=(T // SB,),
        in_specs=[
            pl.BlockSpec((SB, D), lambda i: (i, 0)),
            pl.BlockSpec((1, D), lambda i: (0, 0)),
            pl.BlockSpec((1, D), lambda i: (0, 0)),
            pl.BlockSpec((D, 3 * D), lambda i: (0, 0)),
            pl.BlockSpec((1, 3 * D), lambda i: (0, 0)),
        ],
        out_specs=pl.BlockSpec((SB, 3 * D), lambda i: (i, 0)),
        out_shape=jax.ShapeDtypeStruct((T, 3 * D), jnp.float32),
    )(x, ln_g, ln_b, wqkv, bqkv)

    HP = H // 2
    o = pl.pallas_call(
        functools.partial(_attn_body, qb_size=QB, dh=dh),
        grid=(HP, T // QB),
        in_specs=[
            pl.BlockSpec((QB, 2 * dh), lambda h, q: (q, h)),
            pl.BlockSpec((T, 2 * dh), lambda h, q: (0, HP + h)),
            pl.BlockSpec((T, 2 * dh), lambda h, q: (0, 2 * HP + h)),
        ],
        out_specs=pl.BlockSpec((QB, 2 * dh), lambda h, q: (q, h)),
        out_shape=jax.ShapeDtypeStruct((T, D), jnp.float32),
    )(qkv, qkv, qkv)

    return pl.pallas_call(
        functools.partial(_proj_res_body, res_scale=res_scale),
        grid=(T // SB,),
        in_specs=[
            pl.BlockSpec((SB, D), lambda i: (i, 0)),
            pl.BlockSpec((D, D), lambda i: (0, 0)),
            pl.BlockSpec((1, D), lambda i: (0, 0)),
            pl.BlockSpec((SB, D), lambda i: (i, 0)),
        ],
        out_specs=pl.BlockSpec((SB, D), lambda i: (i, 0)),
        out_shape=jax.ShapeDtypeStruct((T, D), jnp.float32),
    )(o, wo, bo, x)


def kernel(tokens, emb, wqkv, bqkv, wo, bo, ln1g, ln1b, ln2g, ln2b,
           w1, b1, w2, b2, wg, ew1, eb1, ew2, eb2, lnfg, lnfb):
    B, S = tokens.shape
    V, D = emb.shape
    T = B * S
    H = 12
    F = w1.shape[1]
    E = wg.shape[1]
    C = 2 * T // E
    EP = 128
    SB = 256

    tok = tokens.reshape(T).astype(jnp.int32)

    rows = _sc_gather(emb, tok, T, D)
    x0 = pl.pallas_call(
        functools.partial(_scale_body, s=math.sqrt(D)),
        grid=(T // SB,),
        in_specs=[pl.BlockSpec((SB, D), lambda i: (i, 0))],
        out_specs=pl.BlockSpec((SB, D), lambda i: (i, 0)),
        out_shape=jax.ShapeDtypeStruct((T, D), jnp.float32),
    )(rows)

    x = _attention_block(x0, wqkv[0], bqkv[0].reshape(1, 3 * D),
                         wo[0], bo[0].reshape(1, D),
                         ln1g[0].reshape(1, D), ln1b[0].reshape(1, D),
                         1.0, T, D, H)
    x = pl.pallas_call(
        _ffn_body,
        grid=(T // SB,),
        in_specs=[
            pl.BlockSpec((SB, D), lambda i: (i, 0)),
            pl.BlockSpec((1, D), lambda i: (0, 0)),
            pl.BlockSpec((1, D), lambda i: (0, 0)),
            pl.BlockSpec((D, F), lambda i: (0, 0)),
            pl.BlockSpec((1, F), lambda i: (0, 0)),
            pl.BlockSpec((F, D), lambda i: (0, 0)),
            pl.BlockSpec((1, D), lambda i: (0, 0)),
        ],
        out_specs=pl.BlockSpec((SB, D), lambda i: (i, 0)),
        out_shape=jax.ShapeDtypeStruct((T, D), jnp.float32),
    )(x, ln2g[0].reshape(1, D), ln2b[0].reshape(1, D),
      w1, b1.reshape(1, F), w2, b2.reshape(1, D))

    x = _attention_block(x, wqkv[1], bqkv[1].reshape(1, 3 * D),
                         wo[1], bo[1].reshape(1, D),
                         ln1g[1].reshape(1, D), ln1b[1].reshape(1, D),
                         1.0, T, D, H)

    wg_pad = jnp.zeros((D, EP), jnp.float32).at[:, :E].set(wg)
    TPAD = 8
    h_pad, s1s, s2s, s1g, s2g, g1, g2, laux = pl.pallas_call(
        functools.partial(_route_body, T=T, E=E, C=C, EP=EP),
        grid=(),
        in_specs=[
            pl.BlockSpec((T, D), lambda: (0, 0)),
            pl.BlockSpec((1, D), lambda: (0, 0)),
            pl.BlockSpec((1, D), lambda: (0, 0)),
            pl.BlockSpec((D, EP), lambda: (0, 0)),
        ],
        out_specs=[
            pl.BlockSpec((T + TPAD, D), lambda: (0, 0)),
            pl.BlockSpec((T, 1), lambda: (0, 0)),
            pl.BlockSpec((T, 1), lambda: (0, 0)),
            pl.BlockSpec((T, 1), lambda: (0, 0)),
            pl.BlockSpec((T, 1), lambda: (0, 0)),
            pl.BlockSpec((T, 1), lambda: (0, 0)),
            pl.BlockSpec((T, 1), lambda: (0, 0)),
            pl.BlockSpec((1, 1), lambda: (0, 0)),
        ],
        out_shape=[
            jax.ShapeDtypeStruct((T + TPAD, D), jnp.float32),
            jax.ShapeDtypeStruct((T, 1), jnp.int32),
            jax.ShapeDtypeStruct((T, 1), jnp.int32),
            jax.ShapeDtypeStruct((T, 1), jnp.int32),
            jax.ShapeDtypeStruct((T, 1), jnp.int32),
            jax.ShapeDtypeStruct((T, 1), jnp.float32),
            jax.ShapeDtypeStruct((T, 1), jnp.float32),
            jax.ShapeDtypeStruct((1, 1), jnp.float32),
        ],
        scratch_shapes=[
            pltpu.VMEM((T, EP), jnp.float32),
            pltpu.VMEM((T, EP), jnp.float32),
        ],
    )(x, ln2g[1].reshape(1, D), ln2b[1].reshape(1, D), wg_pad)

    nslots = E * C
    slot_to_tok = _sc_invert(
        s1s.reshape(T // 128, 128), s2s.reshape(T // 128, 128),
        jnp.arange(T, dtype=jnp.int32).reshape(T // 128, 128), T, nslots)
    ein = _sc_gather(h_pad, slot_to_tok, nslots, D)

    CB = C
    FB = F // 2
    eo = pl.pallas_call(
        _expert_ffn_body,
        grid=(E, F // FB),
        in_specs=[
            pl.BlockSpec((1, CB, D), lambda e, f: (e, 0, 0)),
            pl.BlockSpec((1, D, FB), lambda e, f: (e, 0, f)),
            pl.BlockSpec((1, 1, FB), lambda e, f: (e, 0, f)),
            pl.BlockSpec((1, FB, D), lambda e, f: (e, f, 0)),
            pl.BlockSpec((1, 1, D), lambda e, f: (e, 0, 0)),
        ],
        out_specs=pl.BlockSpec((1, CB, D), lambda e, f: (e, 0, 0)),
        out_shape=jax.ShapeDtypeStruct((E, CB, D), jnp.float32),
    )(ein.reshape(E, C, D), ew1, eb1.reshape(E, 1, F),
      ew2, eb2.reshape(E, 1, D))

    eo_pad = jnp.concatenate(
        [eo.reshape(nslots, D), jnp.zeros((8, D), jnp.float32)], axis=0)

    eo1 = _sc_gather(eo_pad, s1g.reshape(T), T, D)
    eo2 = _sc_gather(eo_pad, s2g.reshape(T), T, D)

    out = pl.pallas_call(
        _combine_ln_body,
        grid=(T // SB,),
        in_specs=[
            pl.BlockSpec((SB, D), lambda i: (i, 0)),
            pl.BlockSpec((SB, D), lambda i: (i, 0)),
            pl.BlockSpec((SB, D), lambda i: (i, 0)),
            pl.BlockSpec((SB, 1), lambda i: (i, 0)),
            pl.BlockSpec((SB, 1), lambda i: (i, 0)),
            pl.BlockSpec((1, D), lambda i: (0, 0)),
            pl.BlockSpec((1, D), lambda i: (0, 0)),
        ],
        out_specs=pl.BlockSpec((SB, D), lambda i: (i, 0)),
        out_shape=jax.ShapeDtypeStruct((T, D), jnp.float32),
    )(x, eo1, eo2, g1, g2, lnfg.reshape(1, D), lnfb.reshape(1, D))

    return out.reshape(B, S, D), laux[0, 0]
